# baseline (speedup 1.0000x reference)
import dataclasses
import functools

import jax
import jax.numpy as jnp
from jax import lax
from jax.experimental import pallas as pl
from jax.experimental.pallas import tpu as pltpu
from jax.experimental.pallas import tpu_sc as plsc

VOCAB = 100000
HIDDEN = 512
MAX_POS = 512
BATCH = 64
SEQ = 512
EPS = 1e-12

NC = 2
NS = 16
NW = NC * NS
S_PER_W = SEQ // NW
CROWS = 32
NCHUNK = S_PER_W * (BATCH // CROWS)
NGROUP = CROWS // 16
NBUF = 4
INV_H = 1.0 / HIDDEN


def _rsqrt(t):
    i = plsc.bitcast(t, jnp.int32)
    y = plsc.bitcast(jnp.int32(0x5F3759DF) - (i >> 1), jnp.float32)
    for _ in range(3):
        y = y * (1.5 - 0.5 * t * y * y)
    return y


_cparams = pltpu.CompilerParams()
if "needs_layout_passes" in pltpu.CompilerParams.__dataclass_fields__:
    _cparams = dataclasses.replace(_cparams, needs_layout_passes=False)


@functools.partial(
    pl.kernel,
    out_type=jax.ShapeDtypeStruct((BATCH * SEQ, HIDDEN), jnp.float32),
    mesh=plsc.VectorSubcoreMesh(core_axis_name="c", subcore_axis_name="s"),
    compiler_params=_cparams,
    scratch_types=[
        pltpu.VMEM((NBUF, CROWS, HIDDEN), jnp.float32),
        pltpu.VMEM((S_PER_W, HIDDEN), jnp.float32),
        pltpu.VMEM((HIDDEN,), jnp.float32),
        pltpu.VMEM((HIDDEN,), jnp.float32),
        pltpu.VMEM((NCHUNK, CROWS), jnp.int32),
        pltpu.VMEM((NCHUNK, CROWS), jnp.int32),
        pltpu.SemaphoreType.DMA((NBUF,)),
        pltpu.SemaphoreType.DMA((NBUF,)),
    ],
)
def _emb_ln(ids_hbm, word_hbm, pos_hbm, gamma_hbm, beta_hbm, out_hbm,
            rows, posb, gvec, bvec, idxb, destb, gsem, ssem):
    cid = lax.axis_index("c")
    sid = lax.axis_index("s")
    wid = sid * NC + cid
    s0 = wid * S_PER_W

    iota = lax.iota(jnp.int32, 16)

    pltpu.sync_copy(ids_hbm.at[pl.ds(2 * s0, NCHUNK)], idxb)
    pltpu.sync_copy(pos_hbm.at[pl.ds(s0, S_PER_W)], posb)
    pltpu.sync_copy(gamma_hbm, gvec)
    pltpu.sync_copy(beta_hbm, bvec)

    for c in range(NCHUNK):
        s, half = c // 2, c % 2
        for q in range(NGROUP):
            b = 32 * half + 16 * q
            destb[c, pl.ds(16 * q, 16)] = SEQ * (b + iota) + (s0 + s)

    rids = [g * 16 + iota for g in range(NGROUP)]

    def compute(b, c):
        rowsk = rows.at[b]
        cvec = jnp.full((16,), c // 2, jnp.int32)

        def body1(j, carry):
            cj = jnp.full((16,), j, jnp.int32)
            pj = plsc.load_gather(posb, [cvec, cj])
            out = []
            for g in range(NGROUP):
                acc, acc2 = carry[2 * g], carry[2 * g + 1]
                t = plsc.load_gather(rowsk, [rids[g], cj]) + pj
                out += [acc + t, acc2 + t * t]
            return tuple(out)

        zero = jnp.zeros((16,), jnp.float32)
        accs = lax.fori_loop(0, HIDDEN, body1, (zero,) * (2 * NGROUP), unroll=4)
        stats = []
        for g in range(NGROUP):
            mean = accs[2 * g] * INV_H
            var = accs[2 * g + 1] * INV_H - mean * mean
            stats.append((mean, _rsqrt(var + EPS)))

        def body2(j, _):
            cj = jnp.full((16,), j, jnp.int32)
            pj = plsc.load_gather(posb, [cvec, cj])
            gj = plsc.load_gather(gvec, [cj])
            bj = plsc.load_gather(bvec, [cj])
            for g in range(NGROUP):
                mean, rstd = stats[g]
                a = rstd * gj
                off = (pj - mean) * a + bj
                x = plsc.load_gather(rowsk, [rids[g], cj])
                plsc.store_scatter(rowsk, [rids[g], cj], x * a + off)
            return 0

        lax.fori_loop(0, HIDDEN, body2, 0, unroll=4)

    def wait_gather(b):
        pltpu.make_async_copy(word_hbm.at[idxb.at[0]], rows.at[b], gsem.at[b]).wait()

    def wait_scatter(b):
        pltpu.make_async_copy(rows.at[b], out_hbm.at[destb.at[0]], ssem.at[b]).wait()

    for b in range(NBUF):
        pltpu.async_copy(word_hbm.at[idxb.at[b]], rows.at[b], gsem.at[b])

    @pl.loop(0, NCHUNK // NBUF)
    def _(it):
        for b in range(NBUF):
            c = it * NBUF + b
            pb = (b - 1) % NBUF
            wait_gather(b)
            compute(b, c)

            @pl.when(jnp.logical_and(c >= 1, c + NBUF - 1 < NCHUNK))
            def _():
                wait_scatter(pb)
                pltpu.async_copy(word_hbm.at[idxb.at[c + NBUF - 1]], rows.at[pb],
                                 gsem.at[pb])

            pltpu.async_copy(rows.at[b], out_hbm.at[destb.at[c]], ssem.at[b])

    for b in range(NBUF):
        wait_scatter(b)


def kernel(input_ids, word_emb, pos_emb, gamma, beta):
    ids_t = jnp.transpose(input_ids.astype(jnp.int32)).reshape(SEQ * 2, BATCH // 2)
    out = _emb_ln(ids_t, word_emb, pos_emb, gamma, beta)
    return out.reshape(BATCH, SEQ, HIDDEN)
